# baseline (speedup 1.0000x reference)
import jax
import jax.numpy as jnp
from jax import lax
from jax.experimental import pallas as pl
from jax.experimental.pallas import tpu as pltpu
from jax.experimental.pallas import tpu_sc as plsc

BATCH = 4096
SEQ = 200
DIM = 32
OD = 2 * DIM
N = BATCH * SEQ
NC, NS = 2, 16
NW = NC * NS
CHUNK = 128
IDX_BLK = 8
RING = 8
SB_PER_W = SEQ // IDX_BLK


def _body(idx_hbm, w_hbm, out_hbm, idx_v, real_v, imag_v, stage_v,
          sem_i, sem_g, sem_o):
    wr_hbm = w_hbm.at[0]
    wi_hbm = w_hbm.at[1]
    wid = lax.axis_index("s") * NC + lax.axis_index("c")
    b0 = wid * CHUNK

    iota = lax.iota(jnp.int32, 16)
    dvec = [iota, iota + 16]

    def fetch_idx(sb, slot):
        return pltpu.async_copy(
            idx_hbm.at[pl.ds(sb * IDX_BLK, IDX_BLK), pl.ds(b0, CHUNK)],
            idx_v.at[slot], sem_i.at[slot])

    def gather(sb_slot, j):
        pltpu.async_copy(wr_hbm.at[idx_v.at[sb_slot, j]],
                         real_v.at[j], sem_g.at[j])
        pltpu.async_copy(wi_hbm.at[idx_v.at[sb_slot, j]],
                         imag_v.at[j], sem_g.at[j])

    def wait_gather(sb_slot, j):
        pltpu.make_async_copy(wr_hbm.at[idx_v.at[sb_slot, j]],
                              real_v.at[j], sem_g.at[j]).wait()
        pltpu.make_async_copy(wi_hbm.at[idx_v.at[sb_slot, j]],
                              imag_v.at[j], sem_g.at[j]).wait()

    def interleave(j, oslot):
        def toks(bg, _):
            for k in range(2):
                bl = bg * 2 + k
                c0 = jnp.full((16,), 0, jnp.int32) + bl
                c1 = c0 + 128
                for h in range(2):
                    ra = real_v[j, bl, pl.ds(16 * h, 16)]
                    ia = imag_v[j, bl, pl.ds(16 * h, 16)]
                    plsc.store_scatter(stage_v.at[oslot], [dvec[h], c0], ra)
                    plsc.store_scatter(stage_v.at[oslot], [dvec[h], c1], ia)
            return 0
        lax.fori_loop(0, CHUNK // 2, toks, 0, unroll=2)

    def out_slab(s):
        return out_hbm.at[s, :, wid, :]

    def stage_slab(oslot):
        return stage_v.at[oslot, :, pl.ds(0, 2 * CHUNK)]

    def drain_out(s, oslot):
        pltpu.make_async_copy(stage_slab(oslot), out_slab(s),
                              sem_o.at[oslot]).wait()

    fetch_idx(0, 0).wait()
    for j in range(IDX_BLK):
        gather(0, j)
    fetch_idx(1, 1)

    def superblock(sb, _):
        sb_slot = sb & 1
        for j in range(IDX_BLK):
            wait_gather(sb_slot, j)
            oslot = j & 1
            @pl.when(jnp.logical_or(sb > 0, j >= 2))
            def _():
                jm2 = (j - 2) % IDX_BLK
                sbm = jnp.where(j >= 2, sb, sb - 1)
                drain_out(sbm * IDX_BLK + jm2, oslot)

            interleave(j, oslot)

            @pl.when(sb + 1 < SB_PER_W)
            def _():
                if j == 0:
                    pltpu.make_async_copy(
                        idx_hbm.at[pl.ds(0, IDX_BLK), pl.ds(b0, CHUNK)],
                        idx_v.at[1 - sb_slot], sem_i.at[1 - sb_slot]).wait()
                gather(1 - sb_slot, j)

            @pl.when(sb + 2 < SB_PER_W)
            def _():
                if j == IDX_BLK - 1:
                    fetch_idx(sb + 2, sb_slot)

            s = sb * IDX_BLK + j
            pltpu.async_copy(stage_slab(oslot), out_slab(s), sem_o.at[oslot])
        return 0

    lax.fori_loop(0, SB_PER_W, superblock, 0)

    for j in (IDX_BLK - 2, IDX_BLK - 1):
        drain_out((SB_PER_W - 1) * IDX_BLK + j, j & 1)


@jax.jit
def _dembed(idst, w_cat):
    mesh = plsc.VectorSubcoreMesh(core_axis_name="c", subcore_axis_name="s")
    f = pl.kernel(
        _body,
        out_type=jax.ShapeDtypeStruct((SEQ, DIM, NW, 2 * CHUNK), jnp.float32),
        mesh=mesh,
        scratch_types=[
            pltpu.VMEM((2, IDX_BLK, CHUNK), jnp.int32),
            pltpu.VMEM((RING, CHUNK, DIM), jnp.float32),
            pltpu.VMEM((RING, CHUNK, DIM), jnp.float32),
            pltpu.VMEM((2, DIM, 2 * CHUNK + 1), jnp.float32),
            pltpu.SemaphoreType.DMA((2,)),
            pltpu.SemaphoreType.DMA((RING,)),
            pltpu.SemaphoreType.DMA((2,)),
        ],
        compiler_params=pltpu.CompilerParams(
            needs_layout_passes=False, use_tc_tiling_on_sc=False),
    )
    return f(idst, w_cat)


def kernel(token_ids, W_real, W_imag):
    idst = jnp.transpose(token_ids)
    w_cat = jnp.stack([W_real, W_imag])
    x = _dembed(idst, w_cat)
    x = x.reshape(SEQ, DIM, NW, 2, CHUNK)
    x = x.transpose(2, 4, 0, 1, 3)
    return x.reshape(BATCH, SEQ, DIM, 2)


# speedup vs baseline: 1.2229x; 1.2229x over previous
import jax
import jax.numpy as jnp
from jax import lax
from jax.experimental import pallas as pl
from jax.experimental.pallas import tpu as pltpu
from jax.experimental.pallas import tpu_sc as plsc

BATCH = 4096
SEQ = 200
DIM = 32
OD = 2 * DIM
N = BATCH * SEQ
NC, NS = 2, 16
NW = NC * NS
CHUNK = 128
IDX_BLK = 8
RING = 8
SB_PER_W = SEQ // IDX_BLK


def _body(idx_hbm, wr_hbm, wi_hbm, out_hbm, idx_v, real_v, imag_v, stage_v,
          sem_i, sem_g, sem_o):
    wid = lax.axis_index("s") * NC + lax.axis_index("c")
    b0 = wid * CHUNK

    iota = lax.iota(jnp.int32, 16)
    dvec = [iota, iota + 16]

    def fetch_idx(sb, slot):
        return pltpu.async_copy(
            idx_hbm.at[pl.ds(sb * IDX_BLK, IDX_BLK), pl.ds(b0, CHUNK)],
            idx_v.at[slot], sem_i.at[slot])

    def gather(sb_slot, j):
        pltpu.async_copy(wr_hbm.at[idx_v.at[sb_slot, j]],
                         real_v.at[j], sem_g.at[j])
        pltpu.async_copy(wi_hbm.at[idx_v.at[sb_slot, j]],
                         imag_v.at[j], sem_g.at[j])

    def wait_gather(sb_slot, j):
        pltpu.make_async_copy(wr_hbm.at[idx_v.at[sb_slot, j]],
                              real_v.at[j], sem_g.at[j]).wait()
        pltpu.make_async_copy(wi_hbm.at[idx_v.at[sb_slot, j]],
                              imag_v.at[j], sem_g.at[j]).wait()

    def interleave(j, oslot):
        def toks(bg, _):
            for k in range(2):
                bl = bg * 2 + k
                c0 = jnp.full((16,), 0, jnp.int32) + bl
                c1 = c0 + 128
                for h in range(2):
                    ra = real_v[j, bl, pl.ds(16 * h, 16)]
                    ia = imag_v[j, bl, pl.ds(16 * h, 16)]
                    plsc.store_scatter(stage_v.at[oslot], [dvec[h], c0], ra)
                    plsc.store_scatter(stage_v.at[oslot], [dvec[h], c1], ia)
            return 0
        lax.fori_loop(0, CHUNK // 2, toks, 0, unroll=2)

    def out_slab(s):
        return out_hbm.at[s, :, wid, :]

    def stage_slab(oslot):
        return stage_v.at[oslot, :, pl.ds(0, 2 * CHUNK)]

    def drain_out(s, oslot):
        pltpu.make_async_copy(stage_slab(oslot), out_slab(s),
                              sem_o.at[oslot]).wait()

    fetch_idx(0, 0).wait()
    for j in range(IDX_BLK):
        gather(0, j)
    fetch_idx(1, 1)

    def superblock(sb, _):
        sb_slot = sb & 1
        for j in range(IDX_BLK):
            wait_gather(sb_slot, j)
            oslot = j & 1
            @pl.when(jnp.logical_or(sb > 0, j >= 2))
            def _():
                jm2 = (j - 2) % IDX_BLK
                sbm = jnp.where(j >= 2, sb, sb - 1)
                drain_out(sbm * IDX_BLK + jm2, oslot)

            interleave(j, oslot)

            @pl.when(sb + 1 < SB_PER_W)
            def _():
                if j == 0:
                    pltpu.make_async_copy(
                        idx_hbm.at[pl.ds(0, IDX_BLK), pl.ds(b0, CHUNK)],
                        idx_v.at[1 - sb_slot], sem_i.at[1 - sb_slot]).wait()
                gather(1 - sb_slot, j)

            @pl.when(sb + 2 < SB_PER_W)
            def _():
                if j == IDX_BLK - 1:
                    fetch_idx(sb + 2, sb_slot)

            s = sb * IDX_BLK + j
            pltpu.async_copy(stage_slab(oslot), out_slab(s), sem_o.at[oslot])
        return 0

    lax.fori_loop(0, SB_PER_W, superblock, 0)

    for j in (IDX_BLK - 2, IDX_BLK - 1):
        drain_out((SB_PER_W - 1) * IDX_BLK + j, j & 1)


@jax.jit
def _dembed(idst, w_real, w_imag):
    mesh = plsc.VectorSubcoreMesh(core_axis_name="c", subcore_axis_name="s")
    f = pl.kernel(
        _body,
        out_type=jax.ShapeDtypeStruct((SEQ, DIM, NW, 2 * CHUNK), jnp.float32),
        mesh=mesh,
        scratch_types=[
            pltpu.VMEM((2, IDX_BLK, CHUNK), jnp.int32),
            pltpu.VMEM((RING, CHUNK, DIM), jnp.float32),
            pltpu.VMEM((RING, CHUNK, DIM), jnp.float32),
            pltpu.VMEM((2, DIM, 2 * CHUNK + 1), jnp.float32),
            pltpu.SemaphoreType.DMA((2,)),
            pltpu.SemaphoreType.DMA((RING,)),
            pltpu.SemaphoreType.DMA((2,)),
        ],
        compiler_params=pltpu.CompilerParams(
            needs_layout_passes=False, use_tc_tiling_on_sc=False),
    )
    return f(idst, w_real, w_imag)


def kernel(token_ids, W_real, W_imag):
    idst = jnp.transpose(token_ids)
    x = _dembed(idst, W_real, W_imag)
    x = x.reshape(SEQ, DIM, NW, 2, CHUNK)
    x = x.transpose(2, 4, 0, 1, 3)
    return x.reshape(BATCH, SEQ, DIM, 2)
